# baseline (speedup 1.0000x reference)
import functools

import jax
import jax.numpy as jnp
from jax import lax
from jax.experimental import pallas as pl
from jax.experimental.pallas import tpu as pltpu
from jax.experimental.pallas import tpu_sc as plsc

T = 128
K = 8
H = 4096
PERMUTED_ROWS = T * K
EPS = 1e-6
NC, NS = 2, 16
NW = NC * NS
L = 16
NCHUNK = H // L

S_TC = 96
T_SC = T - S_TC
TPW = T_SC // NW


def _rsqrt_vec(x):
    xi = lax.bitcast_convert_type(x, jnp.int32)
    yi = jnp.int32(0x5F3759DF) - (xi >> 1)
    y = lax.bitcast_convert_type(yi, jnp.float32)
    for _ in range(3):
        y = y * (1.5 - 0.5 * x * y * y)
    return y


def _tec_kernel(inp_hbm, idx_hbm, scale_hbm, res_hbm, sh_hbm, w_hbm,
                hid_out, resout_out,
                idx_v, scale_v, g_v, res_v, sh_v, w_v,
                outres_v, hid_v,
                gsem0, gsem1, ldsem, osem0, osem1, hsem0, hsem1):
    wid = lax.axis_index("s") * NC + lax.axis_index("c")
    base = wid * TPW
    src = S_TC + base

    gsem = (gsem0, gsem1)
    osem = (osem0, osem1)
    hsem = (hsem0, hsem1)

    scale_cp = pltpu.async_copy(scale_hbm.at[pl.ds(src * K, TPW * K)],
                                scale_v.at[pl.ds(0, TPW * K)], ldsem)
    pltpu.sync_copy(idx_hbm.at[pl.ds(src, TPW)], idx_v)

    gather = [None, None]
    gather[0] = pltpu.async_copy(inp_hbm.at[idx_v.at[0]], g_v.at[0], gsem[0])
    res_cp = pltpu.async_copy(res_hbm.at[pl.ds(src, TPW)], res_v, ldsem)
    sh_cp = pltpu.async_copy(sh_hbm.at[pl.ds(src, TPW)], sh_v, ldsem)
    w_cp = pltpu.async_copy(w_hbm, w_v, ldsem)

    out_pend = [None, None]

    for t in range(TPW):
        b = t % 2
        gather[b].wait()
        if t + 1 < TPW:
            nb = (t + 1) % 2
            gather[nb] = pltpu.async_copy(
                inp_hbm.at[idx_v.at[t + 1]], g_v.at[nb], gsem[nb])
        if t == 0:
            scale_cp.wait()
            res_cp.wait()
            sh_cp.wait()
            w_cp.wait()
        if out_pend[b] is not None:
            out_pend[b][0].wait()
            out_pend[b][1].wait()

        svec = scale_v[pl.ds((t // 2) * L, L)]
        s = [svec[(t % 2) * K + kk] for kk in range(K)]

        def one_chunk(bb, t=t, b=b, s=s):
            p = [g_v[b, kk, pl.ds(bb, L)] * s[kk] for kk in range(K)]
            q = [p[0] + p[1], p[2] + p[3], p[4] + p[5], p[6] + p[7]]
            r0 = (q[0] + q[1]) + (res_v[t, pl.ds(bb, L)]
                                  + sh_v[t, pl.ds(bb, L)])
            return r0 + (q[2] + q[3])

        @plsc.parallel_loop(0, NCHUNK, step=2,
                            carry=(jnp.zeros((L,), jnp.float32),
                                   jnp.zeros((L,), jnp.float32)))
        def ssq(c, carry, b=b):
            sa, sb = carry
            acc_a = one_chunk(c * L)
            acc_b = one_chunk((c + 1) * L)
            outres_v[b, 0, pl.ds(c * L, L)] = acc_a
            outres_v[b, 0, pl.ds((c + 1) * L, L)] = acc_b
            return (sa + acc_a * acc_a, sb + acc_b * acc_b)

        ssum = ssq[0] + ssq[1]
        outres_cp = pltpu.async_copy(
            outres_v.at[b], resout_out.at[pl.ds(base + t, 1)], osem[b])
        tot = ssum[0]
        for lane in range(1, L):
            tot = tot + ssum[lane]
        rs = _rsqrt_vec(jnp.full((L,), tot * (1.0 / H) + EPS, jnp.float32))

        @plsc.parallel_loop(0, NCHUNK, step=2)
        def _(c, b=b, rs=rs):
            for u in range(2):
                bb = (c + u) * L
                hid_v[b, 0, pl.ds(bb, L)] = (outres_v[b, 0, pl.ds(bb, L)]
                                             * rs * w_v[pl.ds(bb, L)])

        out_pend[b] = (
            outres_cp,
            pltpu.async_copy(hid_v.at[b],
                             hid_out.at[pl.ds(base + t, 1)], hsem[b]),
        )

    for b in range(2):
        if out_pend[b] is not None:
            out_pend[b][0].wait()
            out_pend[b][1].wait()


_sc_finalize = pl.kernel(
    _tec_kernel,
    out_type=(jax.ShapeDtypeStruct((T_SC, H), jnp.float32),
              jax.ShapeDtypeStruct((T_SC, H), jnp.float32)),
    mesh=plsc.VectorSubcoreMesh(core_axis_name="c", subcore_axis_name="s"),
    scratch_types=[
        pltpu.VMEM((TPW, K), jnp.int32),
        pltpu.VMEM((max(TPW * K, L),), jnp.float32),
        pltpu.VMEM((2, K, H), jnp.float32),
        pltpu.VMEM((TPW, H), jnp.float32),
        pltpu.VMEM((TPW, H), jnp.float32),
        pltpu.VMEM((H,), jnp.float32),
        pltpu.VMEM((2, 1, H), jnp.float32),
        pltpu.VMEM((2, 1, H), jnp.float32),
        pltpu.SemaphoreType.DMA,
        pltpu.SemaphoreType.DMA,
        pltpu.SemaphoreType.DMA,
        pltpu.SemaphoreType.DMA,
        pltpu.SemaphoreType.DMA,
        pltpu.SemaphoreType.DMA,
        pltpu.SemaphoreType.DMA,
    ],
)


PCHUNK = 128
NPC = PERMUTED_ROWS // PCHUNK


def _tc_body(idx_ref, scale_ref, tbl_lo_ref, tbl_hi_ref, res_ref, sh_ref,
             w_ref, hid_ref, outres_ref, acc_ref):
    j = pl.program_id(0)
    lo = j * PCHUNK

    pio = lo + lax.broadcasted_iota(jnp.int32, (S_TC, PCHUNK), 1)
    oh = jnp.zeros((S_TC, PCHUNK), jnp.float32)
    for kk in range(K):
        idx_col = idx_ref[:, kk:kk + 1]
        sc_col = scale_ref[:, kk:kk + 1]
        oh = oh + jnp.where(pio == idx_col, sc_col, 0.0)

    part = jnp.concatenate(
        [jnp.dot(oh, tbl_lo_ref[...], preferred_element_type=jnp.float32),
         jnp.dot(oh, tbl_hi_ref[...], preferred_element_type=jnp.float32)],
        axis=1)

    @pl.when(j == 0)
    def _():
        acc_ref[...] = part

    @pl.when(j > 0)
    def _():
        acc_ref[...] = acc_ref[...] + part

    @pl.when(j == NPC - 1)
    def _():
        outr = acc_ref[...] + res_ref[...] + sh_ref[...]
        outres_ref[...] = outr
        var = jnp.mean(outr * outr, axis=1, keepdims=True)
        hid_ref[...] = outr * lax.rsqrt(var + EPS) * w_ref[...]


_tc_finalize = pl.pallas_call(
    _tc_body,
    grid=(NPC,),
    in_specs=[
        pl.BlockSpec((S_TC, K), lambda j: (0, 0)),
        pl.BlockSpec((S_TC, K), lambda j: (0, 0)),
        pl.BlockSpec((PCHUNK, H // 2), lambda j: (j, 0)),
        pl.BlockSpec((PCHUNK, H // 2), lambda j: (j, 1)),
        pl.BlockSpec((S_TC, H), lambda j: (0, 0)),
        pl.BlockSpec((S_TC, H), lambda j: (0, 0)),
        pl.BlockSpec((1, H), lambda j: (0, 0)),
    ],
    out_specs=[
        pl.BlockSpec((S_TC, H), lambda j: (0, 0)),
        pl.BlockSpec((S_TC, H), lambda j: (0, 0)),
    ],
    out_shape=(jax.ShapeDtypeStruct((T, H), jnp.float32),
               jax.ShapeDtypeStruct((T, H), jnp.float32)),
    scratch_shapes=[pltpu.VMEM((S_TC, H), jnp.float32)],
)


def kernel(input, residual, norm_weight, expanded_idx_to_permuted_idx,
           shared_expert_output, expert_scale_factor):
    w2d = norm_weight.reshape(1, H)

    sc_hid, sc_outres = _sc_finalize(
        input, expanded_idx_to_permuted_idx,
        expert_scale_factor.reshape(T * K),
        residual, shared_expert_output, norm_weight)

    tc_hid, tc_outres = _tc_finalize(
        expanded_idx_to_permuted_idx, expert_scale_factor,
        input, input, residual, shared_expert_output, w2d)

    return (lax.dynamic_update_slice(tc_hid, sc_hid, (S_TC, 0)),
            lax.dynamic_update_slice(tc_outres, sc_outres, (S_TC, 0)))


# speedup vs baseline: 1.1012x; 1.1012x over previous
import functools

import jax
import jax.numpy as jnp
from jax import lax
from jax.experimental import pallas as pl
from jax.experimental.pallas import tpu as pltpu
from jax.experimental.pallas import tpu_sc as plsc

T = 128
K = 8
H = 4096
PERMUTED_ROWS = T * K
EPS = 1e-6
NC, NS = 2, 16
NW = NC * NS
L = 16
NCHUNK = H // L

S_TC = 96
T_SC = T - S_TC
TPW = T_SC // NW


def _rsqrt_vec(x):
    xi = lax.bitcast_convert_type(x, jnp.int32)
    yi = jnp.int32(0x5F3759DF) - (xi >> 1)
    y = lax.bitcast_convert_type(yi, jnp.float32)
    for _ in range(3):
        y = y * (1.5 - 0.5 * x * y * y)
    return y


def _tec_kernel(inp_hbm, idx_hbm, scale_hbm, res_hbm, sh_hbm, w_hbm,
                hid_out, resout_out,
                idx_v, scale_v, g_v, res_v, sh_v, w_v,
                outres_v, hid_v,
                gsem0, gsem1, ldsem, osem0, osem1, hsem0, hsem1):
    wid = lax.axis_index("s") * NC + lax.axis_index("c")
    base = wid * TPW
    src = S_TC + base

    gsem = (gsem0, gsem1)
    osem = (osem0, osem1)
    hsem = (hsem0, hsem1)

    scale_cp = pltpu.async_copy(scale_hbm.at[pl.ds(src * K, TPW * K)],
                                scale_v.at[pl.ds(0, TPW * K)], ldsem)
    pltpu.sync_copy(idx_hbm.at[pl.ds(src, TPW)], idx_v)

    gather = [None, None]
    gather[0] = pltpu.async_copy(inp_hbm.at[idx_v.at[0]], g_v.at[0], gsem[0])
    res_cp = pltpu.async_copy(res_hbm.at[pl.ds(src, TPW)], res_v, ldsem)
    sh_cp = pltpu.async_copy(sh_hbm.at[pl.ds(src, TPW)], sh_v, ldsem)
    w_cp = pltpu.async_copy(w_hbm, w_v, ldsem)

    out_pend = [None, None]

    for t in range(TPW):
        b = t % 2
        gather[b].wait()
        if t + 1 < TPW:
            nb = (t + 1) % 2
            gather[nb] = pltpu.async_copy(
                inp_hbm.at[idx_v.at[t + 1]], g_v.at[nb], gsem[nb])
        if t == 0:
            scale_cp.wait()
            res_cp.wait()
            sh_cp.wait()
            w_cp.wait()
        if out_pend[b] is not None:
            out_pend[b][0].wait()
            out_pend[b][1].wait()

        svec = scale_v[pl.ds((t // 2) * L, L)]
        s = [svec[(t % 2) * K + kk] for kk in range(K)]

        def one_chunk(bb, t=t, b=b, s=s):
            p = [g_v[b, kk, pl.ds(bb, L)] * s[kk] for kk in range(K)]
            q = [p[0] + p[1], p[2] + p[3], p[4] + p[5], p[6] + p[7]]
            r0 = (q[0] + q[1]) + (res_v[t, pl.ds(bb, L)]
                                  + sh_v[t, pl.ds(bb, L)])
            return r0 + (q[2] + q[3])

        @plsc.parallel_loop(0, NCHUNK, step=2,
                            carry=(jnp.zeros((L,), jnp.float32),
                                   jnp.zeros((L,), jnp.float32)))
        def ssq(c, carry, b=b):
            sa, sb = carry
            acc_a = one_chunk(c * L)
            acc_b = one_chunk((c + 1) * L)
            outres_v[b, 0, pl.ds(c * L, L)] = acc_a
            outres_v[b, 0, pl.ds((c + 1) * L, L)] = acc_b
            return (sa + acc_a * acc_a, sb + acc_b * acc_b)

        ssum = ssq[0] + ssq[1]
        outres_cp = pltpu.async_copy(
            outres_v.at[b], resout_out.at[pl.ds(base + t, 1)], osem[b])
        tot = ssum[0]
        for lane in range(1, L):
            tot = tot + ssum[lane]
        rs = _rsqrt_vec(jnp.full((L,), tot * (1.0 / H) + EPS, jnp.float32))

        @plsc.parallel_loop(0, NCHUNK, step=2)
        def _(c, b=b, rs=rs):
            for u in range(2):
                bb = (c + u) * L
                hid_v[b, 0, pl.ds(bb, L)] = (outres_v[b, 0, pl.ds(bb, L)]
                                             * rs * w_v[pl.ds(bb, L)])

        out_pend[b] = (
            outres_cp,
            pltpu.async_copy(hid_v.at[b],
                             hid_out.at[pl.ds(base + t, 1)], hsem[b]),
        )

    for b in range(2):
        if out_pend[b] is not None:
            out_pend[b][0].wait()
            out_pend[b][1].wait()


_sc_finalize = pl.kernel(
    _tec_kernel,
    out_type=(jax.ShapeDtypeStruct((T_SC, H), jnp.float32),
              jax.ShapeDtypeStruct((T_SC, H), jnp.float32)),
    mesh=plsc.VectorSubcoreMesh(core_axis_name="c", subcore_axis_name="s"),
    scratch_types=[
        pltpu.VMEM((TPW, K), jnp.int32),
        pltpu.VMEM((max(TPW * K, L),), jnp.float32),
        pltpu.VMEM((2, K, H), jnp.float32),
        pltpu.VMEM((TPW, H), jnp.float32),
        pltpu.VMEM((TPW, H), jnp.float32),
        pltpu.VMEM((H,), jnp.float32),
        pltpu.VMEM((2, 1, H), jnp.float32),
        pltpu.VMEM((2, 1, H), jnp.float32),
        pltpu.SemaphoreType.DMA,
        pltpu.SemaphoreType.DMA,
        pltpu.SemaphoreType.DMA,
        pltpu.SemaphoreType.DMA,
        pltpu.SemaphoreType.DMA,
        pltpu.SemaphoreType.DMA,
        pltpu.SemaphoreType.DMA,
    ],
)


PCHUNK = 256
NPC = PERMUTED_ROWS // PCHUNK


def _tc_body(idx_ref, scale_ref, tbl_lo_ref, tbl_hi_ref, res_ref, sh_ref,
             w_ref, hid_ref, outres_ref, acc_ref):
    j = pl.program_id(0)
    lo = j * PCHUNK

    pio = lo + lax.broadcasted_iota(jnp.int32, (S_TC, PCHUNK), 1)
    oh = jnp.zeros((S_TC, PCHUNK), jnp.float32)
    for kk in range(K):
        idx_col = idx_ref[:, kk:kk + 1]
        sc_col = scale_ref[:, kk:kk + 1]
        oh = oh + jnp.where(pio == idx_col, sc_col, 0.0)

    part = jnp.concatenate(
        [jnp.dot(oh, tbl_lo_ref[...], preferred_element_type=jnp.float32),
         jnp.dot(oh, tbl_hi_ref[...], preferred_element_type=jnp.float32)],
        axis=1)

    @pl.when(j == 0)
    def _():
        acc_ref[...] = part

    @pl.when(j > 0)
    def _():
        acc_ref[...] = acc_ref[...] + part

    @pl.when(j == NPC - 1)
    def _():
        outr = acc_ref[...] + res_ref[...] + sh_ref[...]
        outres_ref[...] = outr
        var = jnp.mean(outr * outr, axis=1, keepdims=True)
        hid_ref[...] = outr * lax.rsqrt(var + EPS) * w_ref[...]


_tc_finalize = pl.pallas_call(
    _tc_body,
    grid=(NPC,),
    in_specs=[
        pl.BlockSpec((S_TC, K), lambda j: (0, 0)),
        pl.BlockSpec((S_TC, K), lambda j: (0, 0)),
        pl.BlockSpec((PCHUNK, H // 2), lambda j: (j, 0)),
        pl.BlockSpec((PCHUNK, H // 2), lambda j: (j, 1)),
        pl.BlockSpec((S_TC, H), lambda j: (0, 0)),
        pl.BlockSpec((S_TC, H), lambda j: (0, 0)),
        pl.BlockSpec((1, H), lambda j: (0, 0)),
    ],
    out_specs=[
        pl.BlockSpec((S_TC, H), lambda j: (0, 0)),
        pl.BlockSpec((S_TC, H), lambda j: (0, 0)),
    ],
    out_shape=(jax.ShapeDtypeStruct((T, H), jnp.float32),
               jax.ShapeDtypeStruct((T, H), jnp.float32)),
    scratch_shapes=[pltpu.VMEM((S_TC, H), jnp.float32)],
)


def kernel(input, residual, norm_weight, expanded_idx_to_permuted_idx,
           shared_expert_output, expert_scale_factor):
    w2d = norm_weight.reshape(1, H)

    sc_hid, sc_outres = _sc_finalize(
        input, expanded_idx_to_permuted_idx,
        expert_scale_factor.reshape(T * K),
        residual, shared_expert_output, norm_weight)

    tc_hid, tc_outres = _tc_finalize(
        expanded_idx_to_permuted_idx, expert_scale_factor,
        input, input, residual, shared_expert_output, w2d)

    return (lax.dynamic_update_slice(tc_hid, sc_hid, (S_TC, 0)),
            lax.dynamic_update_slice(tc_outres, sc_outres, (S_TC, 0)))
